# baseline (speedup 1.0000x reference)
import jax
import jax.numpy as jnp
from jax import lax
from jax.experimental import pallas as pl
from jax.experimental.pallas import tpu as pltpu
from jax.experimental.pallas import tpu_sc as plsc

NUM_FREQ = 3
D = 1024
_NC, _NS = 2, 16
_NW = _NC * _NS
CH = 32
RU = 4


def _sc_body(idx_hbm, x_hbm, emb_hbm, out_hbm, idx_v, row_v, b0, b1,
             s_i0, s_i1, s_o0, s_o1):
    n_rows = x_hbm.shape[0]
    rows_per_worker = n_rows // _NW
    n_chunks = rows_per_worker // CH
    wid = lax.axis_index("s") * _NC + lax.axis_index("c")
    base = wid * rows_per_worker

    pltpu.sync_copy(idx_hbm, idx_v)
    pltpu.async_copy(emb_hbm.at[idx_v], row_v, s_i0).wait()

    pltpu.async_copy(x_hbm.at[pl.ds(base, CH)], b0, s_i0)
    pltpu.async_copy(x_hbm.at[pl.ds(base + CH, CH)], b1, s_i1)

    def process(g, buf, s_in, s_out):
        start = base + g * CH
        pltpu.make_async_copy(x_hbm.at[pl.ds(start, CH)], buf, s_in).wait()

        for jb in range(D // 128):
            rv = [row_v[0, pl.ds((jb * 8 + j) * 16, 16)] for j in range(8)]

            def row_body(r0, carry):
                for u in range(RU):
                    r = r0 * RU + u
                    for j in range(8):
                        sl = pl.ds((jb * 8 + j) * 16, 16)
                        buf[r, sl] = buf[r, sl] + rv[j]
                return carry

            lax.fori_loop(0, CH // RU, row_body, 0)
        pltpu.async_copy(buf, out_hbm.at[pl.ds(start, CH)], s_out)

    def refill(g, buf, s_in, s_out):
        @pl.when(g < n_chunks)
        def _():
            start = base + g * CH
            pltpu.make_async_copy(buf, out_hbm.at[pl.ds(base, CH)], s_out).wait()
            pltpu.async_copy(x_hbm.at[pl.ds(start, CH)], buf, s_in)

    def outer(t, carry):
        g0 = t * 2
        process(g0, b0, s_i0, s_o0)
        process(g0 + 1, b1, s_i1, s_o1)
        refill(g0 + 2, b0, s_i0, s_o0)
        refill(g0 + 3, b1, s_i1, s_o1)
        return carry

    lax.fori_loop(0, n_chunks // 2, outer, 0)
    pltpu.make_async_copy(b0, out_hbm.at[pl.ds(base, CH)], s_o0).wait()
    pltpu.make_async_copy(b1, out_hbm.at[pl.ds(base, CH)], s_o1).wait()


def kernel(x, freq_idx, freq_embeddings):
    orig_shape = x.shape
    d = orig_shape[-1]
    x2 = x.reshape(-1, d)
    n_rows = x2.shape[0]
    idx_arr = jnp.asarray(freq_idx, jnp.int32).reshape(1)

    mesh = plsc.VectorSubcoreMesh(core_axis_name="c", subcore_axis_name="s")
    f = pl.kernel(
        _sc_body,
        out_type=jax.ShapeDtypeStruct((n_rows, d), x.dtype),
        mesh=mesh,
        scratch_types=[
            pltpu.VMEM((1,), jnp.int32),
            pltpu.VMEM((1, d), jnp.float32),
            pltpu.VMEM((CH, d), jnp.float32),
            pltpu.VMEM((CH, d), jnp.float32),
            pltpu.SemaphoreType.DMA,
            pltpu.SemaphoreType.DMA,
            pltpu.SemaphoreType.DMA,
            pltpu.SemaphoreType.DMA,
        ],
    )
    out = f(idx_arr, x2, freq_embeddings)
    return out.reshape(orig_shape)


# speedup vs baseline: 2.5056x; 2.5056x over previous
import jax
import jax.numpy as jnp
from jax import lax
from jax.experimental import pallas as pl
from jax.experimental.pallas import tpu as pltpu
from jax.experimental.pallas import tpu_sc as plsc

NUM_FREQ = 3
ROWS_PER_BLOCK = 2048


def _sc_lookup_body(idx_hbm, emb_hbm, out_hbm, idx_s):
    @pl.when(lax.axis_index("c") == 0)
    def _():
        pltpu.sync_copy(idx_hbm, idx_s)
        idx = idx_s[0]
        pltpu.sync_copy(emb_hbm.at[pl.ds(idx, 1)], out_hbm)


def _sc_lookup(idx_arr, freq_embeddings):
    d = freq_embeddings.shape[-1]
    mesh = plsc.ScalarSubcoreMesh(axis_name="c", num_cores=1)
    f = pl.kernel(
        _sc_lookup_body,
        out_type=jax.ShapeDtypeStruct((1, d), jnp.float32),
        mesh=mesh,
        scratch_types=[
            pltpu.SMEM((1,), jnp.int32),
        ],
    )
    return f(idx_arr, freq_embeddings)


def _tc_add_body(x_ref, row_ref, o_ref):
    o_ref[...] = x_ref[...] + row_ref[...]


def kernel(x, freq_idx, freq_embeddings):
    orig_shape = x.shape
    d = orig_shape[-1]
    x2 = x.reshape(-1, d)
    n_rows = x2.shape[0]
    rpb = ROWS_PER_BLOCK
    idx_arr = jnp.asarray(freq_idx, jnp.int32).reshape(1)

    row = _sc_lookup(idx_arr, freq_embeddings)

    out = pl.pallas_call(
        _tc_add_body,
        grid=(n_rows // rpb,),
        in_specs=[
            pl.BlockSpec((rpb, d), lambda i: (i, 0)),
            pl.BlockSpec((1, d), lambda i: (0, 0)),
        ],
        out_specs=pl.BlockSpec((rpb, d), lambda i: (i, 0)),
        out_shape=jax.ShapeDtypeStruct((n_rows, d), x.dtype),
        compiler_params=pltpu.CompilerParams(
            dimension_semantics=("arbitrary",),
        ),
    )(x2, row)
    return out.reshape(orig_shape)
